# baseline (speedup 1.0000x reference)
import functools

import jax
import jax.numpy as jnp
from jax import lax
from jax.experimental import pallas as pl
from jax.experimental.pallas import tpu as pltpu
from jax.experimental.pallas import tpu_sc as plsc

_NC = 2
_NS = 16
_NW = _NC * _NS


def _sc_gather(embed, rel, idx_flat, B, D):
    cpw = B // _NW
    ngroups = cpw // 16

    mesh = plsc.VectorSubcoreMesh(core_axis_name="c", subcore_axis_name="s")

    @functools.partial(
        pl.kernel,
        mesh=mesh,
        out_type=jax.ShapeDtypeStruct((3, B, D), jnp.float32),
        scratch_types=[
            pltpu.VMEM((cpw,), jnp.int32),
            pltpu.VMEM((cpw, D), jnp.float32),
            pltpu.SemaphoreType.DMA,
        ],
    )
    def gather_kernel(idx_hbm, embed_hbm, rel_hbm, out_hbm,
                      idx_v, rows_v, sem):
        c = lax.axis_index("c")
        s = lax.axis_index("s")
        wid = s * _NC + c
        base = wid * cpw
        for t in range(3):
            table = embed_hbm if t < 2 else rel_hbm
            pltpu.sync_copy(idx_hbm.at[pl.ds(t * B + base, cpw)], idx_v)

            @pl.loop(0, ngroups)
            def _issue(i):
                vec = idx_v[pl.ds(i * 16, 16)]
                for r in range(16):
                    pltpu.async_copy(
                        table.at[pl.ds(vec[r], 1)],
                        rows_v.at[pl.ds(i * 16 + r, 1)],
                        sem,
                    )

            @pl.loop(0, ngroups)
            def _drain(i):
                for r in range(16):
                    pltpu.make_async_copy(
                        table.at[pl.ds(0, 1)],
                        rows_v.at[pl.ds(i * 16 + r, 1)],
                        sem,
                    ).wait()

            pltpu.sync_copy(rows_v, out_hbm.at[t, pl.ds(base, cpw)])

    return gather_kernel(idx_flat, embed, rel)


def _tc_linear(g, wo_t, bo, wr_t, br, B, D):
    bs = 2048
    grid = B // bs

    def body(g_ref, wo_ref, wr_ref, bo_ref, br_ref, o_ref):
        cc = jnp.dot(g_ref[0], wo_ref[:], preferred_element_type=jnp.float32)
        rr = jnp.dot(g_ref[2], wr_ref[:], preferred_element_type=jnp.float32)
        dd = jnp.dot(g_ref[1], wo_ref[:], preferred_element_type=jnp.float32)
        o_ref[:] = jnp.concatenate(
            [cc + bo_ref[:], rr + br_ref[:], dd + bo_ref[:]], axis=-1
        )

    return pl.pallas_call(
        body,
        grid=(grid,),
        in_specs=[
            pl.BlockSpec((3, bs, D), lambda i: (0, i, 0)),
            pl.BlockSpec((D, D), lambda i: (0, 0)),
            pl.BlockSpec((D, D), lambda i: (0, 0)),
            pl.BlockSpec((1, D), lambda i: (0, 0)),
            pl.BlockSpec((1, D), lambda i: (0, 0)),
        ],
        out_specs=pl.BlockSpec((bs, 3 * D), lambda i: (i, 0)),
        out_shape=jax.ShapeDtypeStruct((B, 3 * D), jnp.float32),
    )(g, wo_t, wr_t, bo, br)


def kernel(embed, embed_rel, W_obj, b_obj, W_rel, b_rel, obj_data, rel_data, idx):
    B = obj_data.shape[0]
    D = embed.shape[1]
    idx_flat = jnp.concatenate([obj_data[:, 0], obj_data[:, 1], rel_data])
    g = _sc_gather(embed, embed_rel, idx_flat, B, D)
    return _tc_linear(
        g, W_obj.T, b_obj.reshape(1, D), W_rel.T, b_rel.reshape(1, D), B, D
    )


# speedup vs baseline: 1.2082x; 1.2082x over previous
import functools

import jax
import jax.numpy as jnp
from jax import lax
from jax.experimental import pallas as pl
from jax.experimental.pallas import tpu as pltpu
from jax.experimental.pallas import tpu_sc as plsc

_NC = 2
_NS = 16
_NW = _NC * _NS


def _tc_transpose(embed_t, N, D):
    chunk = 8192
    grid = pl.cdiv(N, chunk)

    def body(in_ref, o_ref):
        o_ref[:] = in_ref[:].T

    return pl.pallas_call(
        body,
        grid=(grid,),
        in_specs=[pl.BlockSpec((D, chunk), lambda i: (0, i))],
        out_specs=pl.BlockSpec((chunk, D), lambda i: (i, 0)),
        out_shape=jax.ShapeDtypeStruct((N, D), jnp.float32),
    )(embed_t)


def _sc_gather(embed, rel, idx_flat, B, D):
    cpw = B // _NW
    ngroups = cpw // 16

    mesh = plsc.VectorSubcoreMesh(core_axis_name="c", subcore_axis_name="s")

    @functools.partial(
        pl.kernel,
        mesh=mesh,
        out_type=jax.ShapeDtypeStruct((3, B, D), jnp.float32),
        scratch_types=[
            pltpu.VMEM((cpw,), jnp.int32),
            pltpu.VMEM((cpw, D), jnp.float32),
            pltpu.SemaphoreType.DMA,
        ],
    )
    def gather_kernel(idx_hbm, embed_hbm, rel_hbm, out_hbm,
                      idx_v, rows_v, sem):
        c = lax.axis_index("c")
        s = lax.axis_index("s")
        wid = s * _NC + c
        base = wid * cpw
        for t in range(3):
            table = embed_hbm if t < 2 else rel_hbm
            pltpu.sync_copy(idx_hbm.at[pl.ds(t * B + base, cpw)], idx_v)

            @pl.loop(0, ngroups)
            def _issue(i):
                vec = idx_v[pl.ds(i * 16, 16)]
                for r in range(16):
                    pltpu.async_copy(
                        table.at[pl.ds(vec[r], 1)],
                        rows_v.at[pl.ds(i * 16 + r, 1)],
                        sem,
                    )

            @pl.loop(0, ngroups)
            def _drain(i):
                for r in range(16):
                    pltpu.make_async_copy(
                        table.at[pl.ds(0, 1)],
                        rows_v.at[pl.ds(i * 16 + r, 1)],
                        sem,
                    ).wait()

            pltpu.sync_copy(rows_v, out_hbm.at[t, pl.ds(base, cpw)])

    return gather_kernel(idx_flat, embed, rel)


def _tc_linear(g, wo_t, bo, wr_t, br, B, D):
    bs = 2048
    grid = B // bs

    def body(g_ref, wo_ref, wr_ref, bo_ref, br_ref, o_ref):
        cc = jnp.dot(g_ref[0], wo_ref[:], preferred_element_type=jnp.float32)
        rr = jnp.dot(g_ref[2], wr_ref[:], preferred_element_type=jnp.float32)
        dd = jnp.dot(g_ref[1], wo_ref[:], preferred_element_type=jnp.float32)
        o_ref[:] = jnp.concatenate(
            [cc + bo_ref[:], rr + br_ref[:], dd + bo_ref[:]], axis=-1
        )

    return pl.pallas_call(
        body,
        grid=(grid,),
        in_specs=[
            pl.BlockSpec((3, bs, D), lambda i: (0, i, 0)),
            pl.BlockSpec((D, D), lambda i: (0, 0)),
            pl.BlockSpec((D, D), lambda i: (0, 0)),
            pl.BlockSpec((1, D), lambda i: (0, 0)),
            pl.BlockSpec((1, D), lambda i: (0, 0)),
        ],
        out_specs=pl.BlockSpec((bs, 3 * D), lambda i: (i, 0)),
        out_shape=jax.ShapeDtypeStruct((B, 3 * D), jnp.float32),
    )(g, wo_t, wr_t, bo, br)


def kernel(embed, embed_rel, W_obj, b_obj, W_rel, b_rel, obj_data, rel_data, idx):
    B = obj_data.shape[0]
    N, D = embed.shape
    idx_flat = jnp.concatenate([obj_data[:, 0], obj_data[:, 1], rel_data])
    embed_rm = _tc_transpose(embed.T, N, D)
    g = _sc_gather(embed_rm, embed_rel, idx_flat, B, D)
    return _tc_linear(
        g, W_obj.T, b_obj.reshape(1, D), W_rel.T, b_rel.reshape(1, D), B, D
    )
